# baseline (speedup 1.0000x reference)
import functools
import math

import jax
import jax.numpy as jnp
from jax import lax
from jax.experimental import pallas as pl
from jax.experimental.pallas import tpu as pltpu
from jax.experimental.pallas import tpu_sc as plsc

D_MODEL = 64
BATCH = 4096
SEQ_LEN = 200
CHUNK = 100
CHUNKS_PER_ROW = SEQ_LEN // CHUNK
NC, NS = 2, 16
NW = NC * NS
ROWS_PER_W = BATCH // NW
NBUF = 2
SCALE = math.sqrt(D_MODEL) / SEQ_LEN


def _pool_kernel(table_hbm, idx_hbm, out_hbm, idx_v, buf0, buf1, out_v,
                 sem0, sem1):
    wid = lax.axis_index("s") * NC + lax.axis_index("c")
    base = wid * ROWS_PER_W
    bufs = (buf0, buf1)
    sems = (sem0, sem1)

    pltpu.sync_copy(
        idx_hbm.at[pl.ds(base * CHUNKS_PER_ROW, ROWS_PER_W * CHUNKS_PER_ROW)],
        idx_v)

    def fire(b, k):
        r = b * CHUNKS_PER_ROW
        d0 = pltpu.async_copy(table_hbm.at[idx_v.at[r]],
                              bufs[k].at[pl.ds(0, CHUNK)], sems[k])
        d1 = pltpu.async_copy(table_hbm.at[idx_v.at[r + 1]],
                              bufs[k].at[pl.ds(CHUNK, CHUNK)], sems[k])
        return d0, d1

    def drain(b, k):
        r = b * CHUNKS_PER_ROW
        pltpu.make_async_copy(table_hbm.at[idx_v.at[r]],
                              bufs[k].at[pl.ds(0, CHUNK)], sems[k]).wait()
        pltpu.make_async_copy(table_hbm.at[idx_v.at[r + 1]],
                              bufs[k].at[pl.ds(CHUNK, CHUNK)], sems[k]).wait()

    def accumulate(b, k):
        buf = bufs[k]

        def body(j, accs):
            return tuple(accs[i] + buf[j, pl.ds(16 * i, 16)]
                         for i in range(4))

        zeros = tuple(jnp.zeros((16,), jnp.float32) for _ in range(4))
        accs = lax.fori_loop(0, SEQ_LEN, body, zeros)
        for i in range(4):
            out_v[b, pl.ds(16 * i, 16)] = accs[i] * SCALE

    for k in range(NBUF):
        fire(k, k)

    def outer(g):
        for k in range(NBUF):
            b = g + k
            drain(b, k)
            accumulate(b, k)
            fire(b + NBUF, k)

    pl.loop(0, ROWS_PER_W - NBUF, step=NBUF)(outer)

    for k in range(NBUF):
        b = ROWS_PER_W - NBUF + k
        drain(b, k)
        accumulate(b, k)

    pltpu.sync_copy(out_v, out_hbm.at[pl.ds(base, ROWS_PER_W)])


@jax.jit
def _pool(src2, embed_weight):
    mesh = plsc.VectorSubcoreMesh(core_axis_name="c", subcore_axis_name="s",
                                  num_cores=NC, num_subcores=NS)
    return pl.kernel(
        _pool_kernel,
        out_type=jax.ShapeDtypeStruct((BATCH, D_MODEL), jnp.float32),
        mesh=mesh,
        scratch_types=[
            pltpu.VMEM((ROWS_PER_W * CHUNKS_PER_ROW, CHUNK), jnp.int32),
            pltpu.VMEM((SEQ_LEN, D_MODEL), jnp.float32),
            pltpu.VMEM((SEQ_LEN, D_MODEL), jnp.float32),
            pltpu.VMEM((ROWS_PER_W, D_MODEL), jnp.float32),
            pltpu.SemaphoreType.DMA,
            pltpu.SemaphoreType.DMA,
        ],
        compiler_params=pltpu.CompilerParams(use_tc_tiling_on_sc=False),
    )(embed_weight, src2)


def kernel(src, embed_weight):
    src2 = src.astype(jnp.int32).reshape(BATCH * CHUNKS_PER_ROW, CHUNK)
    return _pool(src2, embed_weight)
